# Initial kernel scaffold; baseline (speedup 1.0000x reference)
import jax
import jax.numpy as jnp
from jax.experimental import pallas as pl


def kernel(x, kernel):
    raise NotImplementedError("write your pallas kernel here")


# speedup vs baseline: 6.2176x; 6.2176x over previous
import functools

import jax
import jax.numpy as jnp
from jax import lax
from jax.experimental import pallas as pl
from jax.experimental.pallas import tpu as pltpu
from jax.experimental.pallas import tpu_sc as plsc

_D = 32
_B = 16384 * 200
_NC, _NS = 2, 16
_NW = _NC * _NS
_PER_W = _B // _NW
_IDXW = 128
_K = 16
_CHUNK = _K * _IDXW
_NCHUNK = _PER_W // _CHUNK


def _relu_body(t_ref, o_ref):
    o_ref[...] = jnp.maximum(t_ref[...], 0.0)


def _relu_table(table):
    return pl.pallas_call(
        _relu_body,
        out_shape=jax.ShapeDtypeStruct(table.shape, table.dtype),
    )(table)


@functools.partial(
    pl.kernel,
    mesh=plsc.VectorSubcoreMesh(core_axis_name="c", subcore_axis_name="s"),
    compiler_params=pltpu.CompilerParams(use_tc_tiling_on_sc=False),
    out_type=jax.ShapeDtypeStruct((_B, _D), jnp.float32),
    scratch_types=[
        pltpu.VMEM((_K, _IDXW), jnp.int32),
        pltpu.VMEM((_CHUNK, _D), jnp.float32),
        pltpu.SemaphoreType.DMA,
    ],
)
def _sc_gather(table_hbm, idx_hbm, out_hbm, idx_v, rows_v, sem):
    wid = lax.axis_index("s") * _NC + lax.axis_index("c")
    base_irow = wid * (_PER_W // _IDXW)
    base_out = wid * _PER_W

    def body(i, carry):
        pltpu.sync_copy(idx_hbm.at[pl.ds(base_irow + i * _K, _K)], idx_v)
        copies = [
            pltpu.async_copy(
                table_hbm.at[idx_v.at[j]],
                rows_v.at[pl.ds(j * _IDXW, _IDXW)],
                sem,
            )
            for j in range(_K)
        ]
        for c in copies:
            c.wait()
        pltpu.sync_copy(rows_v, out_hbm.at[pl.ds(base_out + i * _CHUNK, _CHUNK)])
        return carry

    lax.fori_loop(0, _NCHUNK, body, 0)


def kernel(x, kernel):
    idx = x.astype(jnp.int32).reshape(_B // _IDXW, _IDXW)
    table = _relu_table(kernel)
    out = _sc_gather(table, idx)
    return out.reshape(x.shape[0], x.shape[1], _D)
